# baseline (speedup 1.0000x reference)
import functools

import jax
import jax.numpy as jnp
from jax import lax
from jax.experimental import pallas as pl
from jax.experimental.pallas import tpu as pltpu
from jax.experimental.pallas import tpu_sc as plsc

_B, _N, _E = 32, 128, 16256
_EPS = 1e-3
_SQRT2 = 1.4142135623730951
_BF = jnp.bfloat16
_LUT = 2048
_CH = 2032
_LO, _HI = -12.0, 12.0

f32 = jnp.float32
i32 = jnp.int32


def _gelu(v):
    return 0.5 * v * (1.0 + jax.lax.erf(v / _SQRT2))


def _bn_scale(p):
    gamma, beta, mm, mv, _, _ = p
    s = gamma / jnp.sqrt(mv + _EPS)
    return s, beta - mm * s


def _bdot(a, w_ref):
    return jnp.dot(a.astype(_BF), w_ref, preferred_element_type=f32)


def _tab_body(hx_ref, bnp_ref, w20_ref, tb_ref, pn_ref, pb_ref):
    hxb = hx_ref[0]
    hxn = hxb * bnp_ref[0:1, 0:20] + bnp_ref[1:2, 0:20]
    hxm = hxb * bnp_ref[2:3, 0:20] + bnp_ref[3:4, 0:20]
    pn_ref[0] = _bdot(hxn, w20_ref[:, 0:16]) + tb_ref[0, 0, 0:16][None, :]
    pb_ref[0] = _bdot(hxm, w20_ref[:, 16:32])


def _rnd_bf16(v):
    u = plsc.bitcast(v, i32)
    r = (u + 0x7FFF + ((u >> 16) & 1)) & jnp.int32(-65536)
    return plsc.bitcast(r, f32)


def _sc_body(pn_hbm, pb_hbm, xt_hbm, nid_hbm, bid_hbm, seg_hbm,
             lutm_hbm, lutc_hbm, wsc_hbm, tbb_hbm, out_hbm,
             pn_v, pb_v, xt_v, nid_v, bid_v, seg_v,
             lutm_v, lutc_v, wsc_v, tbb_v, priv_v):
    wid = lax.axis_index("s") * 2 + lax.axis_index("c")

    pltpu.sync_copy(pn_hbm.at[wid], pn_v)
    pltpu.sync_copy(pb_hbm.at[wid], pb_v)
    pltpu.sync_copy(xt_hbm.at[wid], xt_v)
    pltpu.sync_copy(lutm_hbm, lutm_v)
    pltpu.sync_copy(lutc_hbm, lutc_v)
    pltpu.sync_copy(wsc_hbm, wsc_v)
    pltpu.sync_copy(tbb_hbm.at[wid], tbb_v)

    zeros16 = jnp.zeros((16,), f32)

    def _zero(i):
        priv_v[pl.ds(i * 16, 16)] = zeros16
    plsc.parallel_loop(0, 4096, unroll=4)(_zero)

    def srow(r):
        return wsc_v[pl.ds(r * 16, 16)]

    def splat(vec, k):
        return jnp.full((16,), vec[k], f32)

    w2r = [srow(j) for j in range(16)]
    wc1r = [srow(16 + j) for j in range(16)]
    s2r, t2r = srow(32), srow(33)
    sc1r, tc1r = srow(34), srow(35)
    sc2r, tc2r = srow(36), srow(37)
    war, wbr, wc2r, b2r = srow(38), srow(39), srow(40), srow(41)
    tb0 = tbb_v[pl.ds(0, 16)]
    tb1 = tbb_v[pl.ds(16, 16)]
    w2sp = [[splat(w2r[j], k) for k in range(16)] for j in range(16)]
    wc1sp = [[splat(wc1r[j], k) for k in range(16)] for j in range(16)]
    s2sp = [splat(s2r, j) for j in range(16)]
    t2sp = [splat(t2r, j) for j in range(16)]
    sc1sp = [splat(sc1r, j) for j in range(16)]
    tc1sp = [splat(tc1r, j) for j in range(16)]
    sc2sp = [splat(sc2r, j) for j in range(16)]
    tc2sp = [splat(tc2r, j) for j in range(16)]
    wasp = [splat(war, k) for k in range(16)]
    wbsp = [splat(wbr, k) for k in range(16)]
    wc2sp = [splat(wc2r, k) for k in range(16)]
    b2sp = [splat(b2r, k) for k in range(16)]
    bc1sp = [splat(tb0, k) for k in range(16)]
    casp, cbsp, bcc2sp = splat(tb1, 0), splat(tb1, 1), splat(tb1, 2)

    invstep = jnp.float32((_LUT - 1) / (_HI - _LO))
    off = jnp.float32(-_LO * (_LUT - 1) / (_HI - _LO))

    def glut(v):
        u = v * invstep + off
        u = jnp.minimum(jnp.maximum(u, 0.0), jnp.float32(_LUT - 1))
        idx = u.astype(i32)
        m = plsc.load_gather(lutm_v, [idx])
        c = plsc.load_gather(lutc_v, [idx])
        return m * v + c

    lane = lax.iota(i32, 16)
    ones16 = jnp.ones((16,), f32)

    def _group(g):
        base = g * 16
        node = nid_v[pl.ds(base, 16)]
        nbr = bid_v[pl.ds(base, 16)]
        seg = seg_v[pl.ds(base, 16)]
        n16 = node * 16
        b16 = nbr * 16
        n8 = node * 8
        b8 = nbr * 8

        z1 = [glut(plsc.load_gather(pn_v, [n16 + j])
                   + plsc.load_gather(pb_v, [b16 + j])) for j in range(16)]
        z1s = [_rnd_bf16(z1[j] * s2sp[j] + t2sp[j]) for j in range(16)]
        msg = []
        for k in range(16):
            acc = b2sp[k]
            for j in range(16):
                acc = acc + z1s[j] * w2sp[j][k]
            msg.append(glut(acc))
        msgs = [msg[j] * sc1sp[j] + tc1sp[j] for j in range(16)]
        cfz = []
        for k in range(16):
            acc = bc1sp[k]
            for j in range(16):
                acc = acc + msgs[j] * wc1sp[j][k]
            cfz.append(glut(acc))
        acc_a = casp
        acc_b = cbsp
        acc_c = bcc2sp
        for k in range(16):
            acc_a = acc_a + msg[k] * wasp[k]
            acc_b = acc_b + msg[k] * wbsp[k]
            acc_c = acc_c + (cfz[k] * sc2sp[k] + tc2sp[k]) * wc2sp[k]
        av = glut(acc_a)
        bv = glut(acc_b)
        cfv = glut(acc_c)

        sbase = lane * 4096 + seg * 32
        for k in range(16):
            plsc.addupdate_scatter(priv_v, [sbase + k], msg[k])
        for c in range(4):
            xn = plsc.load_gather(xt_v, [n8 + c])
            xb = plsc.load_gather(xt_v, [b8 + c])
            cu = cfv * (av * xn + bv * xb)
            plsc.addupdate_scatter(priv_v, [sbase + 16 + c], cu)
        plsc.addupdate_scatter(priv_v, [sbase + 20], ones16)

    def _chunk(cc, carry):
        pltpu.sync_copy(nid_hbm.at[wid, cc], nid_v)
        pltpu.sync_copy(bid_hbm.at[wid, cc], bid_v)
        pltpu.sync_copy(seg_hbm.at[cc], seg_v)
        plsc.parallel_loop(0, _CH // 16, unroll=2)(_group)
        return carry

    lax.fori_loop(0, _E // _CH, _chunk, 0)

    def _reduce(n):
        o = n * 32
        acc0 = priv_v[pl.ds(o, 16)]
        acc1 = priv_v[pl.ds(o + 16, 16)]
        for l in range(1, 16):
            acc0 = acc0 + priv_v[pl.ds(l * 4096 + o, 16)]
            acc1 = acc1 + priv_v[pl.ds(l * 4096 + o + 16, 16)]
        priv_v[pl.ds(o, 16)] = acc0
        priv_v[pl.ds(o + 16, 16)] = acc1

    plsc.parallel_loop(0, _N, unroll=2)(_reduce)

    pltpu.sync_copy(priv_v.at[pl.ds(0, 4096)], out_hbm.at[wid])


def _fin_body(sc_ref, hx_ref, bnp_ref, wp_ref, tb_ref, cr_ref,
              ox_ref, oh_ref):
    blk = sc_ref[0]
    aggm = blk[:, 0:16]
    aggc = blk[:, 16:20]
    cnt = blk[:, 20:21]
    hxb = hx_ref[0]
    ox_ref[0] = hxb[:, 16:20] + jnp.where(
        cnt > 0.0, aggc / jnp.maximum(cnt, 1.0), 0.0)
    zi = _gelu(
        _bdot(hxb[:, 0:16] * bnp_ref[10:11, 0:16] + bnp_ref[11:12, 0:16],
              wp_ref[:, 48:64])
        + _bdot(aggm * bnp_ref[12:13, 0:16] + bnp_ref[13:14, 0:16],
                wp_ref[:, 64:80])
        + tb_ref[0, 0, 32:48][None, :])
    oh_ref[0] = _gelu(
        _bdot(zi * bnp_ref[14:15, 0:16] + bnp_ref[15:16, 0:16],
              wp_ref[:, 32:48]) + cr_ref[0, 16:32][None, :])


def kernel(x, h, edges, edge_weights, time_embed, message_params,
           coord_params, inv_params, Wa, ba, Wb, bb):
    del edge_weights
    s1, t1 = _bn_scale(message_params[0])
    s2, t2 = _bn_scale(message_params[1])
    sc1, tc1 = _bn_scale(coord_params[0])
    sc2, tc2 = _bn_scale(coord_params[1])
    si1, ti1 = _bn_scale(inv_params[0])
    si2, ti2 = _bn_scale(inv_params[1])
    w1, b1 = message_params[0][4], message_params[0][5]
    w2, b2 = message_params[1][4], message_params[1][5]
    wc1, bc1 = coord_params[0][4], coord_params[0][5]
    wc2, bc2 = coord_params[1][4], coord_params[1][5]
    wi1, bi1 = inv_params[0][4], inv_params[0][5]
    wi2, bi2 = inv_params[1][4], inv_params[1][5]

    def bdot(a, w):
        return jnp.dot(a.astype(_BF), w.astype(_BF),
                       preferred_element_type=f32)

    te1 = time_embed * s1[40:48] + t1[40:48]
    bias1_t = bdot(te1, w1[40:48]) + b1
    tec1 = time_embed * sc1[16:24] + tc1[16:24]
    biasc1_t = bdot(tec1, wc1[16:24]) + bc1
    tei1 = time_embed * si1[32:40] + ti1[32:40]
    biasi_t = bdot(tei1, wi1[32:40]) + bi1
    ca = bdot(time_embed, Wa[16:24]) + ba
    cb = bdot(time_embed, Wb[16:24]) + bb
    bc2_b = jnp.broadcast_to(bc2[None, :], (_B, 1))
    tb = jnp.concatenate(
        [bias1_t, biasc1_t, biasi_t, ca, cb, bc2_b,
         jnp.zeros((_B, 13), f32)], axis=1).reshape(_B, 1, 64)

    def row20(v):
        return jnp.concatenate([v, jnp.zeros((32 - v.shape[0],), f32)])
    bnp = jnp.stack([
        row20(jnp.concatenate([s1[0:16], s1[32:36]])),
        row20(jnp.concatenate([t1[0:16], t1[32:36]])),
        row20(jnp.concatenate([s1[16:32], s1[36:40]])),
        row20(jnp.concatenate([t1[16:32], t1[36:40]])),
        row20(s2), row20(t2),
        row20(sc1[0:16]), row20(tc1[0:16]),
        row20(sc2), row20(tc2),
        row20(si1[0:16]), row20(ti1[0:16]),
        row20(si1[16:32]), row20(ti1[16:32]),
        row20(si2), row20(ti2)])

    w20 = jnp.concatenate(
        [jnp.concatenate([w1[0:16], w1[32:36]], axis=0),
         jnp.concatenate([w1[16:32], w1[36:40]], axis=0)],
        axis=1).astype(_BF)
    wp = jnp.concatenate(
        [w2, wc1[0:16], wi2, wi1[0:16], wi1[16:32],
         jnp.zeros((16, 16), f32)], axis=1).astype(_BF)
    cr = jnp.concatenate([b2, bi2, jnp.zeros((32,), f32)])[None, :]

    hx = jnp.concatenate([h, x], axis=2)

    pn_t, pb_t = pl.pallas_call(
        _tab_body,
        grid=(_B,),
        in_specs=[
            pl.BlockSpec((1, _N, 20), lambda b: (b, 0, 0)),
            pl.BlockSpec((16, 32), lambda b: (0, 0)),
            pl.BlockSpec((20, 32), lambda b: (0, 0)),
            pl.BlockSpec((1, 1, 64), lambda b: (b, 0, 0)),
        ],
        out_specs=[
            pl.BlockSpec((1, _N, 16), lambda b: (b, 0, 0)),
            pl.BlockSpec((1, _N, 16), lambda b: (b, 0, 0)),
        ],
        out_shape=[
            jax.ShapeDtypeStruct((_B, _N, 16), f32),
            jax.ShapeDtypeStruct((_B, _N, 16), f32),
        ],
    )(hx, bnp, w20, tb)

    rnd = lambda w: w.astype(_BF).astype(f32)
    wsc = jnp.zeros((48, 16), f32)
    wsc = wsc.at[0:16].set(rnd(w2))
    wsc = wsc.at[16:32].set(rnd(wc1[0:16]))
    wsc = wsc.at[32].set(s2)
    wsc = wsc.at[33].set(t2)
    wsc = wsc.at[34].set(sc1[0:16])
    wsc = wsc.at[35].set(tc1[0:16])
    wsc = wsc.at[36].set(sc2)
    wsc = wsc.at[37].set(tc2)
    wsc = wsc.at[38].set(rnd(Wa[0:16, 0]))
    wsc = wsc.at[39].set(rnd(Wb[0:16, 0]))
    wsc = wsc.at[40].set(rnd(wc2[:, 0]))
    wsc = wsc.at[41].set(b2)
    tbb = jnp.concatenate(
        [biasc1_t, ca, cb, jnp.broadcast_to(bc2[None, :], (_B, 1)),
         jnp.zeros((_B, 13), f32)], axis=1)

    grid_pts = jnp.linspace(_LO, _HI, _LUT)
    gvals = _gelu(grid_pts)
    m_seg = (gvals[1:] - gvals[:-1]) / (grid_pts[1:] - grid_pts[:-1])
    c_seg = gvals[:-1] - m_seg * grid_pts[:-1]
    lutm = jnp.concatenate([m_seg, jnp.ones((1,), f32)]).astype(f32)
    lutc = jnp.concatenate([c_seg, jnp.zeros((1,), f32)]).astype(f32)

    xt = jnp.concatenate([x, jnp.zeros((_B, _N, 4), f32)],
                         axis=2).reshape(_B, _N * 8)
    pn_flat = pn_t.reshape(_B, _N * 16)
    pb_flat = pb_t.reshape(_B, _N * 16)
    nid = edges[:, :, 0].reshape(_B, _E // _CH, _CH)
    bid = edges[:, :, 1].reshape(_B, _E // _CH, _CH)
    seg = edges[0, :, 0].reshape(_E // _CH, _CH)

    mesh = plsc.VectorSubcoreMesh(core_axis_name="c", subcore_axis_name="s")
    sc_out = pl.kernel(
        _sc_body,
        mesh=mesh,
        out_type=jax.ShapeDtypeStruct((_B, _N * 32), f32),
        scratch_types=[
            pltpu.VMEM((_N * 16,), f32),
            pltpu.VMEM((_N * 16,), f32),
            pltpu.VMEM((_N * 8,), f32),
            pltpu.VMEM((_CH,), i32),
            pltpu.VMEM((_CH,), i32),
            pltpu.VMEM((_CH,), i32),
            pltpu.VMEM((_LUT,), f32),
            pltpu.VMEM((_LUT,), f32),
            pltpu.VMEM((768,), f32),
            pltpu.VMEM((32,), f32),
            pltpu.VMEM((16 * 4096,), f32),
        ],
        compiler_params=pltpu.CompilerParams(needs_layout_passes=False),
    )(pn_flat, pb_flat, xt, nid, bid, seg, lutm, lutc,
      wsc.reshape(768), tbb)

    sc_res = sc_out.reshape(_B, _N, 32)

    ox, oh = pl.pallas_call(
        _fin_body,
        grid=(_B,),
        in_specs=[
            pl.BlockSpec((1, _N, 32), lambda b: (b, 0, 0)),
            pl.BlockSpec((1, _N, 20), lambda b: (b, 0, 0)),
            pl.BlockSpec((16, 32), lambda b: (0, 0)),
            pl.BlockSpec((16, 96), lambda b: (0, 0)),
            pl.BlockSpec((1, 1, 64), lambda b: (b, 0, 0)),
            pl.BlockSpec((1, 64), lambda b: (0, 0)),
        ],
        out_specs=[
            pl.BlockSpec((1, _N, 4), lambda b: (b, 0, 0)),
            pl.BlockSpec((1, _N, 16), lambda b: (b, 0, 0)),
        ],
        out_shape=[
            jax.ShapeDtypeStruct((_B, _N, 4), f32),
            jax.ShapeDtypeStruct((_B, _N, 16), f32),
        ],
    )(sc_res, hx, bnp, wp, tb, cr)
    return (ox, oh)


# speedup vs baseline: 2.4167x; 2.4167x over previous
import functools

import jax
import jax.numpy as jnp
from jax import lax
from jax.experimental import pallas as pl
from jax.experimental.pallas import tpu as pltpu
from jax.experimental.pallas import tpu_sc as plsc

_B, _N, _E = 32, 128, 16256
_EPS = 1e-3
_SQRT2 = 1.4142135623730951
_BF = jnp.bfloat16
_LUT = 2048
_CH = 2032
_KSC = 3
_EC = 2032
_LO, _HI = -12.0, 12.0

f32 = jnp.float32
i32 = jnp.int32


def _gelu(v):
    return 0.5 * v * (1.0 + jax.lax.erf(v / _SQRT2))


def _bn_scale(p):
    gamma, beta, mm, mv, _, _ = p
    s = gamma / jnp.sqrt(mv + _EPS)
    return s, beta - mm * s


def _bdot(a, w_ref):
    return jnp.dot(a.astype(_BF), w_ref, preferred_element_type=f32)


def _tab_body(hx_ref, bnp_ref, w20_ref, tb_ref, pn_ref, pb_ref):
    hxb = hx_ref[0]
    hxn = hxb * bnp_ref[0:1, 0:20] + bnp_ref[1:2, 0:20]
    hxm = hxb * bnp_ref[2:3, 0:20] + bnp_ref[3:4, 0:20]
    pn_ref[0] = _bdot(hxn, w20_ref[:, 0:16]) + tb_ref[0, 0, 0:16][None, :]
    pb_ref[0] = _bdot(hxm, w20_ref[:, 16:32])


def _rnd_bf16(v):
    u = plsc.bitcast(v, i32)
    r = (u + 0x7FFF + ((u >> 16) & 1)) & jnp.int32(-65536)
    return plsc.bitcast(r, f32)


def _sc_body(pn_hbm, pb_hbm, xt_hbm, nid_hbm, bid_hbm, seg_hbm,
             lutm_hbm, lutc_hbm, wsc_hbm, tbb_hbm, out_hbm,
             pn_v, pb_v, xt_v, nid_v, bid_v, seg_v,
             lutm_v, lutc_v, wsc_v, tbb_v, priv_v):
    wid = lax.axis_index("s") * 2 + lax.axis_index("c")

    pltpu.sync_copy(pn_hbm.at[wid], pn_v)
    pltpu.sync_copy(pb_hbm.at[wid], pb_v)
    pltpu.sync_copy(xt_hbm.at[wid], xt_v)
    pltpu.sync_copy(lutm_hbm, lutm_v)
    pltpu.sync_copy(lutc_hbm, lutc_v)
    pltpu.sync_copy(wsc_hbm, wsc_v)
    pltpu.sync_copy(tbb_hbm.at[wid], tbb_v)

    zeros16 = jnp.zeros((16,), f32)

    def _zero(i):
        priv_v[pl.ds(i * 16, 16)] = zeros16
    plsc.parallel_loop(0, 4096, unroll=4)(_zero)

    def srow(r):
        return wsc_v[pl.ds(r * 16, 16)]

    def splat(vec, k):
        return jnp.full((16,), vec[k], f32)

    w2r = [srow(j) for j in range(16)]
    wc1r = [srow(16 + j) for j in range(16)]
    s2r, t2r = srow(32), srow(33)
    sc1r, tc1r = srow(34), srow(35)
    sc2r, tc2r = srow(36), srow(37)
    war, wbr, wc2r, b2r = srow(38), srow(39), srow(40), srow(41)
    tb0 = tbb_v[pl.ds(0, 16)]
    tb1 = tbb_v[pl.ds(16, 16)]
    w2sp = [[splat(w2r[j], k) for k in range(16)] for j in range(16)]
    wc1sp = [[splat(wc1r[j], k) for k in range(16)] for j in range(16)]
    s2sp = [splat(s2r, j) for j in range(16)]
    t2sp = [splat(t2r, j) for j in range(16)]
    sc1sp = [splat(sc1r, j) for j in range(16)]
    tc1sp = [splat(tc1r, j) for j in range(16)]
    sc2sp = [splat(sc2r, j) for j in range(16)]
    tc2sp = [splat(tc2r, j) for j in range(16)]
    wasp = [splat(war, k) for k in range(16)]
    wbsp = [splat(wbr, k) for k in range(16)]
    wc2sp = [splat(wc2r, k) for k in range(16)]
    b2sp = [splat(b2r, k) for k in range(16)]
    bc1sp = [splat(tb0, k) for k in range(16)]
    casp, cbsp, bcc2sp = splat(tb1, 0), splat(tb1, 1), splat(tb1, 2)

    invstep = jnp.float32((_LUT - 1) / (_HI - _LO))
    off = jnp.float32(-_LO * (_LUT - 1) / (_HI - _LO))

    def glut(v):
        u = v * invstep + off
        u = jnp.minimum(jnp.maximum(u, 0.0), jnp.float32(_LUT - 1))
        idx = u.astype(i32)
        m = plsc.load_gather(lutm_v, [idx])
        c = plsc.load_gather(lutc_v, [idx])
        return m * v + c

    lane = lax.iota(i32, 16)
    ones16 = jnp.ones((16,), f32)

    def _group(g):
        base = g * 16
        node = nid_v[pl.ds(base, 16)]
        nbr = bid_v[pl.ds(base, 16)]
        seg = seg_v[pl.ds(base, 16)]
        n16 = node * 16
        b16 = nbr * 16
        n8 = node * 8
        b8 = nbr * 8

        z1 = [glut(plsc.load_gather(pn_v, [n16 + j])
                   + plsc.load_gather(pb_v, [b16 + j])) for j in range(16)]
        z1s = [_rnd_bf16(z1[j] * s2sp[j] + t2sp[j]) for j in range(16)]
        msg = []
        for k in range(16):
            acc = b2sp[k]
            for j in range(16):
                acc = acc + z1s[j] * w2sp[j][k]
            msg.append(glut(acc))
        msgs = [msg[j] * sc1sp[j] + tc1sp[j] for j in range(16)]
        cfz = []
        for k in range(16):
            acc = bc1sp[k]
            for j in range(16):
                acc = acc + msgs[j] * wc1sp[j][k]
            cfz.append(glut(acc))
        acc_a = casp
        acc_b = cbsp
        acc_c = bcc2sp
        for k in range(16):
            acc_a = acc_a + msg[k] * wasp[k]
            acc_b = acc_b + msg[k] * wbsp[k]
            acc_c = acc_c + (cfz[k] * sc2sp[k] + tc2sp[k]) * wc2sp[k]
        av = glut(acc_a)
        bv = glut(acc_b)
        cfv = glut(acc_c)

        sbase = lane * 4096 + seg * 32
        for k in range(16):
            plsc.addupdate_scatter(priv_v, [sbase + k], msg[k])
        for c in range(4):
            xn = plsc.load_gather(xt_v, [n8 + c])
            xb = plsc.load_gather(xt_v, [b8 + c])
            cu = cfv * (av * xn + bv * xb)
            plsc.addupdate_scatter(priv_v, [sbase + 16 + c], cu)
        plsc.addupdate_scatter(priv_v, [sbase + 20], ones16)

    def _chunk(cc, carry):
        pltpu.sync_copy(nid_hbm.at[wid, cc], nid_v)
        pltpu.sync_copy(bid_hbm.at[wid, cc], bid_v)
        pltpu.sync_copy(seg_hbm.at[cc], seg_v)
        plsc.parallel_loop(0, _CH // 16, unroll=2)(_group)
        return carry

    lax.fori_loop(0, _KSC, _chunk, 0)

    def _reduce(n):
        o = n * 32
        acc0 = priv_v[pl.ds(o, 16)]
        acc1 = priv_v[pl.ds(o + 16, 16)]
        for l in range(1, 16):
            acc0 = acc0 + priv_v[pl.ds(l * 4096 + o, 16)]
            acc1 = acc1 + priv_v[pl.ds(l * 4096 + o + 16, 16)]
        priv_v[pl.ds(o, 16)] = acc0
        priv_v[pl.ds(o + 16, 16)] = acc1

    plsc.parallel_loop(0, _N, unroll=2)(_reduce)

    pltpu.sync_copy(priv_v.at[pl.ds(0, 4096)], out_hbm.at[wid])


def _tcm_body(hx_ref, nidx_ref, bidx_ref, seg_ref, tb_ref, bnp_ref,
              w20_ref, wp_ref, wcat_ref, cr_ref, pacc_ref,
              thi_s, tlo_s, acc_s):
    c = pl.program_id(1)
    nc = pl.num_programs(1)

    @pl.when(c == 0)
    def _init():
        hxb = hx_ref[0]
        hxn = hxb * bnp_ref[0:1, 0:20] + bnp_ref[1:2, 0:20]
        hxm = hxb * bnp_ref[2:3, 0:20] + bnp_ref[3:4, 0:20]
        bias1 = tb_ref[0, 0, 0:16][None, :]
        pn = _bdot(hxn, w20_ref[:, 0:16]) + bias1
        pb = _bdot(hxm, w20_ref[:, 16:32])
        zeros4 = jnp.zeros((_N, 4), f32)
        xb = hxb[:, 16:20]
        tfull = jnp.concatenate(
            [jnp.concatenate([pn, xb, zeros4], axis=1),
             jnp.concatenate([pb, zeros4, xb], axis=1)], axis=0)
        hi = tfull.astype(_BF)
        thi_s[...] = hi
        tlo_s[...] = (tfull - hi.astype(f32)).astype(_BF)
        acc_s[...] = jnp.zeros((_N, 24), f32)

    ids_n = nidx_ref[0, 0, 0, :]
    ids_b = bidx_ref[0, 0, 0, :]
    seg = seg_ref[0, :]

    lane = jax.lax.broadcasted_iota(i32, (_EC, 2 * _N), 1)
    oh = ((ids_n[:, None] == lane) | (ids_b[:, None] == lane)).astype(_BF)
    g = (jnp.dot(oh, thi_s[...], preferred_element_type=f32)
         + jnp.dot(oh, tlo_s[...], preferred_element_type=f32))

    z1 = _gelu(g[:, 0:16])
    msg = _gelu(_bdot(z1 * bnp_ref[4:5, 0:16] + bnp_ref[5:6, 0:16],
                      wp_ref[:, 0:16]) + cr_ref[0, 0:16][None, :])
    cfz = _gelu(_bdot(msg * bnp_ref[6:7, 0:16] + bnp_ref[7:8, 0:16],
                      wp_ref[:, 16:32]) + tb_ref[0, 0, 16:32][None, :])
    mc = jnp.concatenate(
        [msg, cfz * bnp_ref[8:9, 0:16] + bnp_ref[9:10, 0:16]], axis=1)
    abc = _gelu(_bdot(mc, wcat_ref[...])
                + tb_ref[0, 0, 48:56][None, :])
    cu = abc[:, 2:3] * (abc[:, 0:1] * g[:, 16:20]
                        + abc[:, 1:2] * g[:, 20:24])

    sub = jax.lax.broadcasted_iota(i32, (_N, _EC), 0)
    oh_s = (sub == seg).astype(_BF)
    scat = jnp.concatenate(
        [msg, cu, jnp.ones((_EC, 1), f32), jnp.zeros((_EC, 3), f32)],
        axis=1)
    shi = scat.astype(_BF)
    slo = (scat - shi.astype(f32)).astype(_BF)
    acc_s[...] += (jnp.dot(oh_s, shi, preferred_element_type=f32)
                   + jnp.dot(oh_s, slo, preferred_element_type=f32))

    @pl.when(c == nc - 1)
    def _fin():
        pacc_ref[0] = acc_s[...]


def _fin_body(sc_ref, pacc_ref, hx_ref, bnp_ref, wp_ref, tb_ref, cr_ref,
              ox_ref, oh_ref):
    blk = sc_ref[0]
    pac = pacc_ref[0]
    aggm = blk[:, 0:16] + pac[:, 0:16]
    aggc = blk[:, 16:20] + pac[:, 16:20]
    cnt = blk[:, 20:21] + pac[:, 20:21]
    hxb = hx_ref[0]
    ox_ref[0] = hxb[:, 16:20] + jnp.where(
        cnt > 0.0, aggc / jnp.maximum(cnt, 1.0), 0.0)
    zi = _gelu(
        _bdot(hxb[:, 0:16] * bnp_ref[10:11, 0:16] + bnp_ref[11:12, 0:16],
              wp_ref[:, 48:64])
        + _bdot(aggm * bnp_ref[12:13, 0:16] + bnp_ref[13:14, 0:16],
                wp_ref[:, 64:80])
        + tb_ref[0, 0, 32:48][None, :])
    oh_ref[0] = _gelu(
        _bdot(zi * bnp_ref[14:15, 0:16] + bnp_ref[15:16, 0:16],
              wp_ref[:, 32:48]) + cr_ref[0, 16:32][None, :])


def kernel(x, h, edges, edge_weights, time_embed, message_params,
           coord_params, inv_params, Wa, ba, Wb, bb):
    del edge_weights
    s1, t1 = _bn_scale(message_params[0])
    s2, t2 = _bn_scale(message_params[1])
    sc1, tc1 = _bn_scale(coord_params[0])
    sc2, tc2 = _bn_scale(coord_params[1])
    si1, ti1 = _bn_scale(inv_params[0])
    si2, ti2 = _bn_scale(inv_params[1])
    w1, b1 = message_params[0][4], message_params[0][5]
    w2, b2 = message_params[1][4], message_params[1][5]
    wc1, bc1 = coord_params[0][4], coord_params[0][5]
    wc2, bc2 = coord_params[1][4], coord_params[1][5]
    wi1, bi1 = inv_params[0][4], inv_params[0][5]
    wi2, bi2 = inv_params[1][4], inv_params[1][5]

    def bdot(a, w):
        return jnp.dot(a.astype(_BF), w.astype(_BF),
                       preferred_element_type=f32)

    te1 = time_embed * s1[40:48] + t1[40:48]
    bias1_t = bdot(te1, w1[40:48]) + b1
    tec1 = time_embed * sc1[16:24] + tc1[16:24]
    biasc1_t = bdot(tec1, wc1[16:24]) + bc1
    tei1 = time_embed * si1[32:40] + ti1[32:40]
    biasi_t = bdot(tei1, wi1[32:40]) + bi1
    ca = bdot(time_embed, Wa[16:24]) + ba
    cb = bdot(time_embed, Wb[16:24]) + bb
    bc2_b = jnp.broadcast_to(bc2[None, :], (_B, 1))
    tb = jnp.concatenate(
        [bias1_t, biasc1_t, biasi_t, ca, cb, bc2_b,
         jnp.zeros((_B, 13), f32)], axis=1).reshape(_B, 1, 64)

    def row20(v):
        return jnp.concatenate([v, jnp.zeros((32 - v.shape[0],), f32)])
    bnp = jnp.stack([
        row20(jnp.concatenate([s1[0:16], s1[32:36]])),
        row20(jnp.concatenate([t1[0:16], t1[32:36]])),
        row20(jnp.concatenate([s1[16:32], s1[36:40]])),
        row20(jnp.concatenate([t1[16:32], t1[36:40]])),
        row20(s2), row20(t2),
        row20(sc1[0:16]), row20(tc1[0:16]),
        row20(sc2), row20(tc2),
        row20(si1[0:16]), row20(ti1[0:16]),
        row20(si1[16:32]), row20(ti1[16:32]),
        row20(si2), row20(ti2)])

    w20 = jnp.concatenate(
        [jnp.concatenate([w1[0:16], w1[32:36]], axis=0),
         jnp.concatenate([w1[16:32], w1[36:40]], axis=0)],
        axis=1).astype(_BF)
    wp = jnp.concatenate(
        [w2, wc1[0:16], wi2, wi1[0:16], wi1[16:32],
         jnp.zeros((16, 16), f32)], axis=1).astype(_BF)
    cr = jnp.concatenate([b2, bi2, jnp.zeros((32,), f32)])[None, :]
    z16 = jnp.zeros((16, 1), f32)
    wcat = jnp.concatenate(
        [jnp.concatenate([Wa[0:16], Wb[0:16], z16], axis=1),
         jnp.concatenate([z16, z16, wc2], axis=1)], axis=0)
    wcat = jnp.concatenate([wcat, jnp.zeros((32, 5), f32)],
                           axis=1).astype(_BF)

    hx = jnp.concatenate([h, x], axis=2)

    pn_t, pb_t = pl.pallas_call(
        _tab_body,
        grid=(_B,),
        in_specs=[
            pl.BlockSpec((1, _N, 20), lambda b: (b, 0, 0)),
            pl.BlockSpec((16, 32), lambda b: (0, 0)),
            pl.BlockSpec((20, 32), lambda b: (0, 0)),
            pl.BlockSpec((1, 1, 64), lambda b: (b, 0, 0)),
        ],
        out_specs=[
            pl.BlockSpec((1, _N, 16), lambda b: (b, 0, 0)),
            pl.BlockSpec((1, _N, 16), lambda b: (b, 0, 0)),
        ],
        out_shape=[
            jax.ShapeDtypeStruct((_B, _N, 16), f32),
            jax.ShapeDtypeStruct((_B, _N, 16), f32),
        ],
    )(hx, bnp, w20, tb)

    rnd = lambda w: w.astype(_BF).astype(f32)
    wsc = jnp.zeros((48, 16), f32)
    wsc = wsc.at[0:16].set(rnd(w2))
    wsc = wsc.at[16:32].set(rnd(wc1[0:16]))
    wsc = wsc.at[32].set(s2)
    wsc = wsc.at[33].set(t2)
    wsc = wsc.at[34].set(sc1[0:16])
    wsc = wsc.at[35].set(tc1[0:16])
    wsc = wsc.at[36].set(sc2)
    wsc = wsc.at[37].set(tc2)
    wsc = wsc.at[38].set(rnd(Wa[0:16, 0]))
    wsc = wsc.at[39].set(rnd(Wb[0:16, 0]))
    wsc = wsc.at[40].set(rnd(wc2[:, 0]))
    wsc = wsc.at[41].set(b2)
    tbb = jnp.concatenate(
        [biasc1_t, ca, cb, jnp.broadcast_to(bc2[None, :], (_B, 1)),
         jnp.zeros((_B, 13), f32)], axis=1)

    grid_pts = jnp.linspace(_LO, _HI, _LUT)
    gvals = _gelu(grid_pts)
    m_seg = (gvals[1:] - gvals[:-1]) / (grid_pts[1:] - grid_pts[:-1])
    c_seg = gvals[:-1] - m_seg * grid_pts[:-1]
    lutm = jnp.concatenate([m_seg, jnp.ones((1,), f32)]).astype(f32)
    lutc = jnp.concatenate([c_seg, jnp.zeros((1,), f32)]).astype(f32)

    xt = jnp.concatenate([x, jnp.zeros((_B, _N, 4), f32)],
                         axis=2).reshape(_B, _N * 8)
    pn_flat = pn_t.reshape(_B, _N * 16)
    pb_flat = pb_t.reshape(_B, _N * 16)
    nid = edges[:, :, 0].reshape(_B, _E // _CH, _CH)
    bid = edges[:, :, 1].reshape(_B, _E // _CH, _CH)
    seg = edges[0, :, 0].reshape(_E // _CH, _CH)

    mesh = plsc.VectorSubcoreMesh(core_axis_name="c", subcore_axis_name="s")
    sc_out = pl.kernel(
        _sc_body,
        mesh=mesh,
        out_type=jax.ShapeDtypeStruct((_B, _N * 32), f32),
        scratch_types=[
            pltpu.VMEM((_N * 16,), f32),
            pltpu.VMEM((_N * 16,), f32),
            pltpu.VMEM((_N * 8,), f32),
            pltpu.VMEM((_CH,), i32),
            pltpu.VMEM((_CH,), i32),
            pltpu.VMEM((_CH,), i32),
            pltpu.VMEM((_LUT,), f32),
            pltpu.VMEM((_LUT,), f32),
            pltpu.VMEM((768,), f32),
            pltpu.VMEM((32,), f32),
            pltpu.VMEM((16 * 4096,), f32),
        ],
        compiler_params=pltpu.CompilerParams(needs_layout_passes=False),
    )(pn_flat, pb_flat, xt, nid, bid, seg, lutm, lutc,
      wsc.reshape(768), tbb)

    sc_res = sc_out.reshape(_B, _N, 32)

    nidx4 = edges[:, :, 0].reshape(_B, _E // _EC, 1, _EC)
    bidx4 = (edges[:, :, 1] + _N).reshape(_B, _E // _EC, 1, _EC)
    seg3 = edges[0, :, 0].reshape(_E // _EC, 1, _EC)
    nc_tc = _E // _EC - _KSC
    pacc = pl.pallas_call(
        _tcm_body,
        grid=(_B, nc_tc),
        in_specs=[
            pl.BlockSpec((1, _N, 20), lambda b, c: (b, 0, 0)),
            pl.BlockSpec((1, 1, 1, _EC), lambda b, c: (b, c + _KSC, 0, 0)),
            pl.BlockSpec((1, 1, 1, _EC), lambda b, c: (b, c + _KSC, 0, 0)),
            pl.BlockSpec((1, 1, _EC), lambda b, c: (c + _KSC, 0, 0)),
            pl.BlockSpec((1, 1, 64), lambda b, c: (b, 0, 0)),
            pl.BlockSpec((16, 32), lambda b, c: (0, 0)),
            pl.BlockSpec((20, 32), lambda b, c: (0, 0)),
            pl.BlockSpec((16, 96), lambda b, c: (0, 0)),
            pl.BlockSpec((32, 8), lambda b, c: (0, 0)),
            pl.BlockSpec((1, 64), lambda b, c: (0, 0)),
        ],
        out_specs=[pl.BlockSpec((1, _N, 24), lambda b, c: (b, 0, 0))],
        out_shape=[jax.ShapeDtypeStruct((_B, _N, 24), f32)],
        scratch_shapes=[
            pltpu.VMEM((2 * _N, 24), _BF),
            pltpu.VMEM((2 * _N, 24), _BF),
            pltpu.VMEM((_N, 24), f32),
        ],
        compiler_params=pltpu.CompilerParams(
            dimension_semantics=("arbitrary", "arbitrary")),
    )(hx, nidx4, bidx4, seg3, tb, bnp, w20, wp, wcat, cr)[0]

    ox, oh = pl.pallas_call(
        _fin_body,
        grid=(_B,),
        in_specs=[
            pl.BlockSpec((1, _N, 32), lambda b: (b, 0, 0)),
            pl.BlockSpec((1, _N, 24), lambda b: (b, 0, 0)),
            pl.BlockSpec((1, _N, 20), lambda b: (b, 0, 0)),
            pl.BlockSpec((16, 32), lambda b: (0, 0)),
            pl.BlockSpec((16, 96), lambda b: (0, 0)),
            pl.BlockSpec((1, 1, 64), lambda b: (b, 0, 0)),
            pl.BlockSpec((1, 64), lambda b: (0, 0)),
        ],
        out_specs=[
            pl.BlockSpec((1, _N, 4), lambda b: (b, 0, 0)),
            pl.BlockSpec((1, _N, 16), lambda b: (b, 0, 0)),
        ],
        out_shape=[
            jax.ShapeDtypeStruct((_B, _N, 4), f32),
            jax.ShapeDtypeStruct((_B, _N, 16), f32),
        ],
    )(sc_res, pacc, hx, bnp, wp, tb, cr)
    return (ox, oh)


# speedup vs baseline: 2.7175x; 1.1245x over previous
import functools

import jax
import jax.numpy as jnp
from jax import lax
from jax.experimental import pallas as pl
from jax.experimental.pallas import tpu as pltpu
from jax.experimental.pallas import tpu_sc as plsc

_B, _N, _E = 32, 128, 16256
_EPS = 1e-3
_SQRT2 = 1.4142135623730951
_BF = jnp.bfloat16
_LUT = 2048
_CH = 2032
_KSC = 2
_EC = 2032
_LO, _HI = -12.0, 12.0

f32 = jnp.float32
i32 = jnp.int32


def _gelu(v):
    return 0.5 * v * (1.0 + jax.lax.erf(v / _SQRT2))


def _bn_scale(p):
    gamma, beta, mm, mv, _, _ = p
    s = gamma / jnp.sqrt(mv + _EPS)
    return s, beta - mm * s


def _bdot(a, w_ref):
    return jnp.dot(a.astype(_BF), w_ref, preferred_element_type=f32)


def _tab_body(hx_ref, bnp_ref, w20_ref, tb_ref, pn_ref, pb_ref):
    hxb = hx_ref[0]
    hxn = hxb * bnp_ref[0:1, 0:20] + bnp_ref[1:2, 0:20]
    hxm = hxb * bnp_ref[2:3, 0:20] + bnp_ref[3:4, 0:20]
    pn_ref[0] = _bdot(hxn, w20_ref[:, 0:16]) + tb_ref[0, 0, 0:16][None, :]
    pb_ref[0] = _bdot(hxm, w20_ref[:, 16:32])


def _rnd_bf16(v):
    u = plsc.bitcast(v, i32)
    r = (u + 0x7FFF + ((u >> 16) & 1)) & jnp.int32(-65536)
    return plsc.bitcast(r, f32)


def _sc_body(pn_hbm, pb_hbm, xt_hbm, nid_hbm, bid_hbm, seg_hbm,
             lutm_hbm, lutc_hbm, wsc_hbm, tbb_hbm, out_hbm,
             pn_v, pb_v, xt_v, nid_v, bid_v, seg_v,
             lutm_v, lutc_v, wsc_v, tbb_v, priv_v):
    wid = lax.axis_index("s") * 2 + lax.axis_index("c")

    pltpu.sync_copy(pn_hbm.at[wid], pn_v)
    pltpu.sync_copy(pb_hbm.at[wid], pb_v)
    pltpu.sync_copy(xt_hbm.at[wid], xt_v)
    pltpu.sync_copy(lutm_hbm, lutm_v)
    pltpu.sync_copy(lutc_hbm, lutc_v)
    pltpu.sync_copy(wsc_hbm, wsc_v)
    pltpu.sync_copy(tbb_hbm.at[wid], tbb_v)

    zeros16 = jnp.zeros((16,), f32)

    def _zero(i):
        priv_v[pl.ds(i * 16, 16)] = zeros16
    plsc.parallel_loop(0, 4096, unroll=4)(_zero)

    def srow(r):
        return wsc_v[pl.ds(r * 16, 16)]

    def splat(vec, k):
        return jnp.full((16,), vec[k], f32)

    w2r = [srow(j) for j in range(16)]
    wc1r = [srow(16 + j) for j in range(16)]
    s2r, t2r = srow(32), srow(33)
    sc1r, tc1r = srow(34), srow(35)
    sc2r, tc2r = srow(36), srow(37)
    war, wbr, wc2r, b2r = srow(38), srow(39), srow(40), srow(41)
    tb0 = tbb_v[pl.ds(0, 16)]
    tb1 = tbb_v[pl.ds(16, 16)]
    w2sp = [[splat(w2r[j], k) for k in range(16)] for j in range(16)]
    wc1sp = [[splat(wc1r[j], k) for k in range(16)] for j in range(16)]
    s2sp = [splat(s2r, j) for j in range(16)]
    t2sp = [splat(t2r, j) for j in range(16)]
    sc1sp = [splat(sc1r, j) for j in range(16)]
    tc1sp = [splat(tc1r, j) for j in range(16)]
    sc2sp = [splat(sc2r, j) for j in range(16)]
    tc2sp = [splat(tc2r, j) for j in range(16)]
    wasp = [splat(war, k) for k in range(16)]
    wbsp = [splat(wbr, k) for k in range(16)]
    wc2sp = [splat(wc2r, k) for k in range(16)]
    b2sp = [splat(b2r, k) for k in range(16)]
    bc1sp = [splat(tb0, k) for k in range(16)]
    casp, cbsp, bcc2sp = splat(tb1, 0), splat(tb1, 1), splat(tb1, 2)

    invstep = jnp.float32((_LUT - 1) / (_HI - _LO))
    off = jnp.float32(-_LO * (_LUT - 1) / (_HI - _LO))

    def glut(v):
        u = v * invstep + off
        u = jnp.minimum(jnp.maximum(u, 0.0), jnp.float32(_LUT - 1))
        idx = u.astype(i32)
        m = plsc.load_gather(lutm_v, [idx])
        c = plsc.load_gather(lutc_v, [idx])
        return m * v + c

    lane = lax.iota(i32, 16)
    ones16 = jnp.ones((16,), f32)

    def _group(g):
        base = g * 16
        node = nid_v[pl.ds(base, 16)]
        nbr = bid_v[pl.ds(base, 16)]
        seg = seg_v[pl.ds(base, 16)]
        n16 = node * 16
        b16 = nbr * 16
        n8 = node * 8
        b8 = nbr * 8

        z1 = [glut(plsc.load_gather(pn_v, [n16 + j])
                   + plsc.load_gather(pb_v, [b16 + j])) for j in range(16)]
        z1s = [_rnd_bf16(z1[j] * s2sp[j] + t2sp[j]) for j in range(16)]
        msg = []
        for k in range(16):
            acc = b2sp[k]
            for j in range(16):
                acc = acc + z1s[j] * w2sp[j][k]
            msg.append(glut(acc))
        msgs = [msg[j] * sc1sp[j] + tc1sp[j] for j in range(16)]
        cfz = []
        for k in range(16):
            acc = bc1sp[k]
            for j in range(16):
                acc = acc + msgs[j] * wc1sp[j][k]
            cfz.append(glut(acc))
        acc_a = casp
        acc_b = cbsp
        acc_c = bcc2sp
        for k in range(16):
            acc_a = acc_a + msg[k] * wasp[k]
            acc_b = acc_b + msg[k] * wbsp[k]
            acc_c = acc_c + (cfz[k] * sc2sp[k] + tc2sp[k]) * wc2sp[k]
        av = glut(acc_a)
        bv = glut(acc_b)
        cfv = glut(acc_c)

        sbase = lane * 4096 + seg * 32
        for k in range(16):
            plsc.addupdate_scatter(priv_v, [sbase + k], msg[k])
        for c in range(4):
            xn = plsc.load_gather(xt_v, [n8 + c])
            xb = plsc.load_gather(xt_v, [b8 + c])
            cu = cfv * (av * xn + bv * xb)
            plsc.addupdate_scatter(priv_v, [sbase + 16 + c], cu)
        plsc.addupdate_scatter(priv_v, [sbase + 20], ones16)

    def _chunk(cc, carry):
        pltpu.sync_copy(nid_hbm.at[wid, cc], nid_v)
        pltpu.sync_copy(bid_hbm.at[wid, cc], bid_v)
        pltpu.sync_copy(seg_hbm.at[cc], seg_v)
        plsc.parallel_loop(0, _CH // 16, unroll=2)(_group)
        return carry

    lax.fori_loop(0, _KSC, _chunk, 0)

    def _reduce(n):
        o = n * 32
        acc0 = priv_v[pl.ds(o, 16)]
        acc1 = priv_v[pl.ds(o + 16, 16)]
        for l in range(1, 16):
            acc0 = acc0 + priv_v[pl.ds(l * 4096 + o, 16)]
            acc1 = acc1 + priv_v[pl.ds(l * 4096 + o + 16, 16)]
        priv_v[pl.ds(o, 16)] = acc0
        priv_v[pl.ds(o + 16, 16)] = acc1

    plsc.parallel_loop(0, _N, unroll=2)(_reduce)

    pltpu.sync_copy(priv_v.at[pl.ds(0, 4096)], out_hbm.at[wid])


def _tcm_body(hx_ref, nidx_ref, bidx_ref, seg_ref, tb_ref, bnp_ref,
              w20_ref, wp_ref, wcat_ref, cr_ref, pacc_ref,
              thi_s, tlo_s, acc_s):
    c = pl.program_id(1)
    nc = pl.num_programs(1)

    @pl.when(c == 0)
    def _init():
        hxb = hx_ref[0]
        hxn = hxb * bnp_ref[0:1, 0:20] + bnp_ref[1:2, 0:20]
        hxm = hxb * bnp_ref[2:3, 0:20] + bnp_ref[3:4, 0:20]
        bias1 = tb_ref[0, 0, 0:16][None, :]
        pn = _bdot(hxn, w20_ref[:, 0:16]) + bias1
        pb = _bdot(hxm, w20_ref[:, 16:32])
        zeros4 = jnp.zeros((_N, 4), f32)
        xb = hxb[:, 16:20]
        tfull = jnp.concatenate(
            [jnp.concatenate([pn, xb, zeros4], axis=1),
             jnp.concatenate([pb, zeros4, xb], axis=1)], axis=0)
        hi = tfull.astype(_BF)
        thi_s[...] = hi
        tlo_s[...] = (tfull - hi.astype(f32)).astype(_BF)
        acc_s[...] = jnp.zeros((_N, 24), f32)

    ids_n = nidx_ref[0, 0, 0, :]
    ids_b = bidx_ref[0, 0, 0, :]
    seg = seg_ref[0, :]

    lane = jax.lax.broadcasted_iota(i32, (_EC, 2 * _N), 1)
    oh = ((ids_n[:, None] == lane) | (ids_b[:, None] == lane)).astype(_BF)
    g = (jnp.dot(oh, thi_s[...], preferred_element_type=f32)
         + jnp.dot(oh, tlo_s[...], preferred_element_type=f32))

    z1 = _gelu(g[:, 0:16])
    msg = _gelu(_bdot(z1 * bnp_ref[4:5, 0:16] + bnp_ref[5:6, 0:16],
                      wp_ref[:, 0:16]) + cr_ref[0, 0:16][None, :])
    cfz = _gelu(_bdot(msg * bnp_ref[6:7, 0:16] + bnp_ref[7:8, 0:16],
                      wp_ref[:, 16:32]) + tb_ref[0, 0, 16:32][None, :])
    mc = jnp.concatenate(
        [msg, cfz * bnp_ref[8:9, 0:16] + bnp_ref[9:10, 0:16]], axis=1)
    abc = _gelu(_bdot(mc, wcat_ref[...])
                + tb_ref[0, 0, 48:56][None, :])
    cu = abc[:, 2:3] * (abc[:, 0:1] * g[:, 16:20]
                        + abc[:, 1:2] * g[:, 20:24])

    sub = jax.lax.broadcasted_iota(i32, (_N, _EC), 0)
    oh_s = (sub == seg).astype(_BF)
    scat = jnp.concatenate(
        [msg, cu, jnp.ones((_EC, 1), f32), jnp.zeros((_EC, 3), f32)],
        axis=1)
    shi = scat.astype(_BF)
    slo = (scat - shi.astype(f32)).astype(_BF)
    acc_s[...] += (jnp.dot(oh_s, shi, preferred_element_type=f32)
                   + jnp.dot(oh_s, slo, preferred_element_type=f32))

    @pl.when(c == nc - 1)
    def _fin():
        pacc_ref[0] = acc_s[...]


def _fin_body(sc_ref, pacc_ref, hx_ref, bnp_ref, wp_ref, tb_ref, cr_ref,
              ox_ref, oh_ref):
    blk = sc_ref[0]
    pac = pacc_ref[0]
    aggm = blk[:, 0:16] + pac[:, 0:16]
    aggc = blk[:, 16:20] + pac[:, 16:20]
    cnt = blk[:, 20:21] + pac[:, 20:21]
    hxb = hx_ref[0]
    ox_ref[0] = hxb[:, 16:20] + jnp.where(
        cnt > 0.0, aggc / jnp.maximum(cnt, 1.0), 0.0)
    zi = _gelu(
        _bdot(hxb[:, 0:16] * bnp_ref[10:11, 0:16] + bnp_ref[11:12, 0:16],
              wp_ref[:, 48:64])
        + _bdot(aggm * bnp_ref[12:13, 0:16] + bnp_ref[13:14, 0:16],
                wp_ref[:, 64:80])
        + tb_ref[0, 0, 32:48][None, :])
    oh_ref[0] = _gelu(
        _bdot(zi * bnp_ref[14:15, 0:16] + bnp_ref[15:16, 0:16],
              wp_ref[:, 32:48]) + cr_ref[0, 16:32][None, :])


def kernel(x, h, edges, edge_weights, time_embed, message_params,
           coord_params, inv_params, Wa, ba, Wb, bb):
    del edge_weights
    s1, t1 = _bn_scale(message_params[0])
    s2, t2 = _bn_scale(message_params[1])
    sc1, tc1 = _bn_scale(coord_params[0])
    sc2, tc2 = _bn_scale(coord_params[1])
    si1, ti1 = _bn_scale(inv_params[0])
    si2, ti2 = _bn_scale(inv_params[1])
    w1, b1 = message_params[0][4], message_params[0][5]
    w2, b2 = message_params[1][4], message_params[1][5]
    wc1, bc1 = coord_params[0][4], coord_params[0][5]
    wc2, bc2 = coord_params[1][4], coord_params[1][5]
    wi1, bi1 = inv_params[0][4], inv_params[0][5]
    wi2, bi2 = inv_params[1][4], inv_params[1][5]

    def bdot(a, w):
        return jnp.dot(a.astype(_BF), w.astype(_BF),
                       preferred_element_type=f32)

    te1 = time_embed * s1[40:48] + t1[40:48]
    bias1_t = bdot(te1, w1[40:48]) + b1
    tec1 = time_embed * sc1[16:24] + tc1[16:24]
    biasc1_t = bdot(tec1, wc1[16:24]) + bc1
    tei1 = time_embed * si1[32:40] + ti1[32:40]
    biasi_t = bdot(tei1, wi1[32:40]) + bi1
    ca = bdot(time_embed, Wa[16:24]) + ba
    cb = bdot(time_embed, Wb[16:24]) + bb
    bc2_b = jnp.broadcast_to(bc2[None, :], (_B, 1))
    tb = jnp.concatenate(
        [bias1_t, biasc1_t, biasi_t, ca, cb, bc2_b,
         jnp.zeros((_B, 13), f32)], axis=1).reshape(_B, 1, 64)

    def row20(v):
        return jnp.concatenate([v, jnp.zeros((32 - v.shape[0],), f32)])
    bnp = jnp.stack([
        row20(jnp.concatenate([s1[0:16], s1[32:36]])),
        row20(jnp.concatenate([t1[0:16], t1[32:36]])),
        row20(jnp.concatenate([s1[16:32], s1[36:40]])),
        row20(jnp.concatenate([t1[16:32], t1[36:40]])),
        row20(s2), row20(t2),
        row20(sc1[0:16]), row20(tc1[0:16]),
        row20(sc2), row20(tc2),
        row20(si1[0:16]), row20(ti1[0:16]),
        row20(si1[16:32]), row20(ti1[16:32]),
        row20(si2), row20(ti2)])

    w20 = jnp.concatenate(
        [jnp.concatenate([w1[0:16], w1[32:36]], axis=0),
         jnp.concatenate([w1[16:32], w1[36:40]], axis=0)],
        axis=1).astype(_BF)
    wp = jnp.concatenate(
        [w2, wc1[0:16], wi2, wi1[0:16], wi1[16:32],
         jnp.zeros((16, 16), f32)], axis=1).astype(_BF)
    cr = jnp.concatenate([b2, bi2, jnp.zeros((32,), f32)])[None, :]
    z16 = jnp.zeros((16, 1), f32)
    wcat = jnp.concatenate(
        [jnp.concatenate([Wa[0:16], Wb[0:16], z16], axis=1),
         jnp.concatenate([z16, z16, wc2], axis=1)], axis=0)
    wcat = jnp.concatenate([wcat, jnp.zeros((32, 5), f32)],
                           axis=1).astype(_BF)

    hx = jnp.concatenate([h, x], axis=2)

    pn_t, pb_t = pl.pallas_call(
        _tab_body,
        grid=(_B,),
        in_specs=[
            pl.BlockSpec((1, _N, 20), lambda b: (b, 0, 0)),
            pl.BlockSpec((16, 32), lambda b: (0, 0)),
            pl.BlockSpec((20, 32), lambda b: (0, 0)),
            pl.BlockSpec((1, 1, 64), lambda b: (b, 0, 0)),
        ],
        out_specs=[
            pl.BlockSpec((1, _N, 16), lambda b: (b, 0, 0)),
            pl.BlockSpec((1, _N, 16), lambda b: (b, 0, 0)),
        ],
        out_shape=[
            jax.ShapeDtypeStruct((_B, _N, 16), f32),
            jax.ShapeDtypeStruct((_B, _N, 16), f32),
        ],
    )(hx, bnp, w20, tb)

    rnd = lambda w: w.astype(_BF).astype(f32)
    wsc = jnp.zeros((48, 16), f32)
    wsc = wsc.at[0:16].set(rnd(w2))
    wsc = wsc.at[16:32].set(rnd(wc1[0:16]))
    wsc = wsc.at[32].set(s2)
    wsc = wsc.at[33].set(t2)
    wsc = wsc.at[34].set(sc1[0:16])
    wsc = wsc.at[35].set(tc1[0:16])
    wsc = wsc.at[36].set(sc2)
    wsc = wsc.at[37].set(tc2)
    wsc = wsc.at[38].set(rnd(Wa[0:16, 0]))
    wsc = wsc.at[39].set(rnd(Wb[0:16, 0]))
    wsc = wsc.at[40].set(rnd(wc2[:, 0]))
    wsc = wsc.at[41].set(b2)
    tbb = jnp.concatenate(
        [biasc1_t, ca, cb, jnp.broadcast_to(bc2[None, :], (_B, 1)),
         jnp.zeros((_B, 13), f32)], axis=1)

    grid_pts = jnp.linspace(_LO, _HI, _LUT)
    gvals = _gelu(grid_pts)
    m_seg = (gvals[1:] - gvals[:-1]) / (grid_pts[1:] - grid_pts[:-1])
    c_seg = gvals[:-1] - m_seg * grid_pts[:-1]
    lutm = jnp.concatenate([m_seg, jnp.ones((1,), f32)]).astype(f32)
    lutc = jnp.concatenate([c_seg, jnp.zeros((1,), f32)]).astype(f32)

    xt = jnp.concatenate([x, jnp.zeros((_B, _N, 4), f32)],
                         axis=2).reshape(_B, _N * 8)
    pn_flat = pn_t.reshape(_B, _N * 16)
    pb_flat = pb_t.reshape(_B, _N * 16)
    nid = edges[:, :, 0].reshape(_B, _E // _CH, _CH)
    bid = edges[:, :, 1].reshape(_B, _E // _CH, _CH)
    seg = edges[0, :, 0].reshape(_E // _CH, _CH)

    mesh = plsc.VectorSubcoreMesh(core_axis_name="c", subcore_axis_name="s")
    sc_out = pl.kernel(
        _sc_body,
        mesh=mesh,
        out_type=jax.ShapeDtypeStruct((_B, _N * 32), f32),
        scratch_types=[
            pltpu.VMEM((_N * 16,), f32),
            pltpu.VMEM((_N * 16,), f32),
            pltpu.VMEM((_N * 8,), f32),
            pltpu.VMEM((_CH,), i32),
            pltpu.VMEM((_CH,), i32),
            pltpu.VMEM((_CH,), i32),
            pltpu.VMEM((_LUT,), f32),
            pltpu.VMEM((_LUT,), f32),
            pltpu.VMEM((768,), f32),
            pltpu.VMEM((32,), f32),
            pltpu.VMEM((16 * 4096,), f32),
        ],
        compiler_params=pltpu.CompilerParams(needs_layout_passes=False),
    )(pn_flat, pb_flat, xt, nid, bid, seg, lutm, lutc,
      wsc.reshape(768), tbb)

    sc_res = sc_out.reshape(_B, _N, 32)

    nidx4 = edges[:, :, 0].reshape(_B, _E // _EC, 1, _EC)
    bidx4 = (edges[:, :, 1] + _N).reshape(_B, _E // _EC, 1, _EC)
    seg3 = edges[0, :, 0].reshape(_E // _EC, 1, _EC)
    nc_tc = _E // _EC - _KSC
    pacc = pl.pallas_call(
        _tcm_body,
        grid=(_B, nc_tc),
        in_specs=[
            pl.BlockSpec((1, _N, 20), lambda b, c: (b, 0, 0)),
            pl.BlockSpec((1, 1, 1, _EC), lambda b, c: (b, c + _KSC, 0, 0)),
            pl.BlockSpec((1, 1, 1, _EC), lambda b, c: (b, c + _KSC, 0, 0)),
            pl.BlockSpec((1, 1, _EC), lambda b, c: (c + _KSC, 0, 0)),
            pl.BlockSpec((1, 1, 64), lambda b, c: (b, 0, 0)),
            pl.BlockSpec((16, 32), lambda b, c: (0, 0)),
            pl.BlockSpec((20, 32), lambda b, c: (0, 0)),
            pl.BlockSpec((16, 96), lambda b, c: (0, 0)),
            pl.BlockSpec((32, 8), lambda b, c: (0, 0)),
            pl.BlockSpec((1, 64), lambda b, c: (0, 0)),
        ],
        out_specs=[pl.BlockSpec((1, _N, 24), lambda b, c: (b, 0, 0))],
        out_shape=[jax.ShapeDtypeStruct((_B, _N, 24), f32)],
        scratch_shapes=[
            pltpu.VMEM((2 * _N, 24), _BF),
            pltpu.VMEM((2 * _N, 24), _BF),
            pltpu.VMEM((_N, 24), f32),
        ],
        compiler_params=pltpu.CompilerParams(
            dimension_semantics=("arbitrary", "arbitrary")),
    )(hx, nidx4, bidx4, seg3, tb, bnp, w20, wp, wcat, cr)[0]

    ox, oh = pl.pallas_call(
        _fin_body,
        grid=(_B,),
        in_specs=[
            pl.BlockSpec((1, _N, 32), lambda b: (b, 0, 0)),
            pl.BlockSpec((1, _N, 24), lambda b: (b, 0, 0)),
            pl.BlockSpec((1, _N, 20), lambda b: (b, 0, 0)),
            pl.BlockSpec((16, 32), lambda b: (0, 0)),
            pl.BlockSpec((16, 96), lambda b: (0, 0)),
            pl.BlockSpec((1, 1, 64), lambda b: (b, 0, 0)),
            pl.BlockSpec((1, 64), lambda b: (0, 0)),
        ],
        out_specs=[
            pl.BlockSpec((1, _N, 4), lambda b: (b, 0, 0)),
            pl.BlockSpec((1, _N, 16), lambda b: (b, 0, 0)),
        ],
        out_shape=[
            jax.ShapeDtypeStruct((_B, _N, 4), f32),
            jax.ShapeDtypeStruct((_B, _N, 16), f32),
        ],
    )(sc_res, pacc, hx, bnp, wp, tb, cr)
    return (ox, oh)


# speedup vs baseline: 2.9343x; 1.0798x over previous
import functools

import jax
import jax.numpy as jnp
from jax import lax
from jax.experimental import pallas as pl
from jax.experimental.pallas import tpu as pltpu
from jax.experimental.pallas import tpu_sc as plsc

_B, _N, _E = 32, 128, 16256
_EPS = 1e-3
_SQRT2 = 1.4142135623730951
_BF = jnp.bfloat16
_LUT = 2048
_CH = 2032
_KSC = 2
_EC = 4064
_KT = 1
_LO, _HI = -12.0, 12.0

f32 = jnp.float32
i32 = jnp.int32


def _gelu(v):
    return 0.5 * v * (1.0 + jax.lax.erf(v / _SQRT2))


def _bn_scale(p):
    gamma, beta, mm, mv, _, _ = p
    s = gamma / jnp.sqrt(mv + _EPS)
    return s, beta - mm * s


def _bdot(a, w_ref):
    return jnp.dot(a.astype(_BF), w_ref, preferred_element_type=f32)


def _tab_body(hx_ref, bnp_ref, w20_ref, tb_ref, pn_ref, pb_ref):
    hxb = hx_ref[0]
    hxn = hxb * bnp_ref[0:1, 0:20] + bnp_ref[1:2, 0:20]
    hxm = hxb * bnp_ref[2:3, 0:20] + bnp_ref[3:4, 0:20]
    pn_ref[0] = _bdot(hxn, w20_ref[:, 0:16]) + tb_ref[0, 0, 0:16][None, :]
    pb_ref[0] = _bdot(hxm, w20_ref[:, 16:32])


def _rnd_bf16(v):
    u = plsc.bitcast(v, i32)
    r = (u + 0x7FFF + ((u >> 16) & 1)) & jnp.int32(-65536)
    return plsc.bitcast(r, f32)


def _sc_body(pn_hbm, pb_hbm, xt_hbm, nid_hbm, bid_hbm, seg_hbm,
             lutm_hbm, lutc_hbm, wsc_hbm, tbb_hbm, out_hbm,
             pn_v, pb_v, xt_v, nid_v, bid_v, seg_v,
             lutm_v, lutc_v, wsc_v, tbb_v, priv_v):
    wid = lax.axis_index("s") * 2 + lax.axis_index("c")

    pltpu.sync_copy(pn_hbm.at[wid], pn_v)
    pltpu.sync_copy(pb_hbm.at[wid], pb_v)
    pltpu.sync_copy(xt_hbm.at[wid], xt_v)
    pltpu.sync_copy(lutm_hbm, lutm_v)
    pltpu.sync_copy(lutc_hbm, lutc_v)
    pltpu.sync_copy(wsc_hbm, wsc_v)
    pltpu.sync_copy(tbb_hbm.at[wid], tbb_v)

    zeros16 = jnp.zeros((16,), f32)

    def _zero(i):
        priv_v[pl.ds(i * 16, 16)] = zeros16
    plsc.parallel_loop(0, 4096, unroll=4)(_zero)

    def srow(r):
        return wsc_v[pl.ds(r * 16, 16)]

    def splat(vec, k):
        return jnp.full((16,), vec[k], f32)

    w2r = [srow(j) for j in range(16)]
    wc1r = [srow(16 + j) for j in range(16)]
    s2r, t2r = srow(32), srow(33)
    sc1r, tc1r = srow(34), srow(35)
    sc2r, tc2r = srow(36), srow(37)
    war, wbr, wc2r, b2r = srow(38), srow(39), srow(40), srow(41)
    tb0 = tbb_v[pl.ds(0, 16)]
    tb1 = tbb_v[pl.ds(16, 16)]
    w2sp = [[splat(w2r[j], k) for k in range(16)] for j in range(16)]
    wc1sp = [[splat(wc1r[j], k) for k in range(16)] for j in range(16)]
    s2sp = [splat(s2r, j) for j in range(16)]
    t2sp = [splat(t2r, j) for j in range(16)]
    sc1sp = [splat(sc1r, j) for j in range(16)]
    tc1sp = [splat(tc1r, j) for j in range(16)]
    sc2sp = [splat(sc2r, j) for j in range(16)]
    tc2sp = [splat(tc2r, j) for j in range(16)]
    wasp = [splat(war, k) for k in range(16)]
    wbsp = [splat(wbr, k) for k in range(16)]
    wc2sp = [splat(wc2r, k) for k in range(16)]
    b2sp = [splat(b2r, k) for k in range(16)]
    bc1sp = [splat(tb0, k) for k in range(16)]
    casp, cbsp, bcc2sp = splat(tb1, 0), splat(tb1, 1), splat(tb1, 2)

    invstep = jnp.float32((_LUT - 1) / (_HI - _LO))
    off = jnp.float32(-_LO * (_LUT - 1) / (_HI - _LO))

    def glut(v):
        u = v * invstep + off
        u = jnp.minimum(jnp.maximum(u, 0.0), jnp.float32(_LUT - 1))
        idx = u.astype(i32)
        m = plsc.load_gather(lutm_v, [idx])
        c = plsc.load_gather(lutc_v, [idx])
        return m * v + c

    lane = lax.iota(i32, 16)
    ones16 = jnp.ones((16,), f32)

    def _group(g):
        base = g * 16
        node = nid_v[pl.ds(base, 16)]
        nbr = bid_v[pl.ds(base, 16)]
        seg = seg_v[pl.ds(base, 16)]
        n16 = node * 16
        b16 = nbr * 16
        n8 = node * 8
        b8 = nbr * 8

        z1 = [glut(plsc.load_gather(pn_v, [n16 + j])
                   + plsc.load_gather(pb_v, [b16 + j])) for j in range(16)]
        z1s = [_rnd_bf16(z1[j] * s2sp[j] + t2sp[j]) for j in range(16)]
        msg = []
        for k in range(16):
            acc = b2sp[k]
            for j in range(16):
                acc = acc + z1s[j] * w2sp[j][k]
            msg.append(glut(acc))
        msgs = [msg[j] * sc1sp[j] + tc1sp[j] for j in range(16)]
        cfz = []
        for k in range(16):
            acc = bc1sp[k]
            for j in range(16):
                acc = acc + msgs[j] * wc1sp[j][k]
            cfz.append(glut(acc))
        acc_a = casp
        acc_b = cbsp
        acc_c = bcc2sp
        for k in range(16):
            acc_a = acc_a + msg[k] * wasp[k]
            acc_b = acc_b + msg[k] * wbsp[k]
            acc_c = acc_c + (cfz[k] * sc2sp[k] + tc2sp[k]) * wc2sp[k]
        av = glut(acc_a)
        bv = glut(acc_b)
        cfv = glut(acc_c)

        sbase = lane * 4096 + seg * 32
        for k in range(16):
            plsc.addupdate_scatter(priv_v, [sbase + k], msg[k])
        for c in range(4):
            xn = plsc.load_gather(xt_v, [n8 + c])
            xb = plsc.load_gather(xt_v, [b8 + c])
            cu = cfv * (av * xn + bv * xb)
            plsc.addupdate_scatter(priv_v, [sbase + 16 + c], cu)
        plsc.addupdate_scatter(priv_v, [sbase + 20], ones16)

    def _chunk(cc, carry):
        pltpu.sync_copy(nid_hbm.at[wid, cc], nid_v)
        pltpu.sync_copy(bid_hbm.at[wid, cc], bid_v)
        pltpu.sync_copy(seg_hbm.at[cc], seg_v)
        plsc.parallel_loop(0, _CH // 16, unroll=2)(_group)
        return carry

    lax.fori_loop(0, _KSC, _chunk, 0)

    def _reduce(n):
        o = n * 32
        acc0 = priv_v[pl.ds(o, 16)]
        acc1 = priv_v[pl.ds(o + 16, 16)]
        for l in range(1, 16):
            acc0 = acc0 + priv_v[pl.ds(l * 4096 + o, 16)]
            acc1 = acc1 + priv_v[pl.ds(l * 4096 + o + 16, 16)]
        priv_v[pl.ds(o, 16)] = acc0
        priv_v[pl.ds(o + 16, 16)] = acc1

    plsc.parallel_loop(0, _N, unroll=2)(_reduce)

    pltpu.sync_copy(priv_v.at[pl.ds(0, 4096)], out_hbm.at[wid])


def _tcm_body(hx_ref, nidx_ref, bidx_ref, seg_ref, tb_ref, bnp_ref,
              w20_ref, wp_ref, wcat_ref, cr_ref, pacc_ref,
              thi_s, tlo_s, acc_s):
    c = pl.program_id(1)
    nc = pl.num_programs(1)

    @pl.when(c == 0)
    def _init():
        hxb = hx_ref[0]
        hxn = hxb * bnp_ref[0:1, 0:20] + bnp_ref[1:2, 0:20]
        hxm = hxb * bnp_ref[2:3, 0:20] + bnp_ref[3:4, 0:20]
        bias1 = tb_ref[0, 0, 0:16][None, :]
        pn = _bdot(hxn, w20_ref[:, 0:16]) + bias1
        pb = _bdot(hxm, w20_ref[:, 16:32])
        zeros4 = jnp.zeros((_N, 4), f32)
        xb = hxb[:, 16:20]
        tfull = jnp.concatenate(
            [jnp.concatenate([pn, xb, zeros4], axis=1),
             jnp.concatenate([pb, zeros4, xb], axis=1)], axis=0)
        hi = tfull.astype(_BF)
        thi_s[...] = hi
        tlo_s[...] = (tfull - hi.astype(f32)).astype(_BF)
        acc_s[...] = jnp.zeros((_N, 24), f32)

    ids_n = nidx_ref[0, 0, 0, :]
    ids_b = bidx_ref[0, 0, 0, :]
    seg = seg_ref[0, :]

    lane = jax.lax.broadcasted_iota(i32, (_EC, 2 * _N), 1)
    oh = ((ids_n[:, None] == lane) | (ids_b[:, None] == lane)).astype(_BF)
    g = (jnp.dot(oh, thi_s[...], preferred_element_type=f32)
         + jnp.dot(oh, tlo_s[...], preferred_element_type=f32))

    z1 = _gelu(g[:, 0:16])
    msg = _gelu(_bdot(z1 * bnp_ref[4:5, 0:16] + bnp_ref[5:6, 0:16],
                      wp_ref[:, 0:16]) + cr_ref[0, 0:16][None, :])
    cfz = _gelu(_bdot(msg * bnp_ref[6:7, 0:16] + bnp_ref[7:8, 0:16],
                      wp_ref[:, 16:32]) + tb_ref[0, 0, 16:32][None, :])
    mc = jnp.concatenate(
        [msg, cfz * bnp_ref[8:9, 0:16] + bnp_ref[9:10, 0:16]], axis=1)
    abc = _gelu(_bdot(mc, wcat_ref[...])
                + tb_ref[0, 0, 48:56][None, :])
    cu = abc[:, 2:3] * (abc[:, 0:1] * g[:, 16:20]
                        + abc[:, 1:2] * g[:, 20:24])

    sub = jax.lax.broadcasted_iota(i32, (_N, _EC), 0)
    oh_s = (sub == seg).astype(_BF)
    scat = jnp.concatenate(
        [msg, cu, jnp.ones((_EC, 1), f32), jnp.zeros((_EC, 3), f32)],
        axis=1)
    shi = scat.astype(_BF)
    slo = (scat - shi.astype(f32)).astype(_BF)
    acc_s[...] += (jnp.dot(oh_s, shi, preferred_element_type=f32)
                   + jnp.dot(oh_s, slo, preferred_element_type=f32))

    @pl.when(c == nc - 1)
    def _fin():
        pacc_ref[0] = acc_s[...]


def _fin_body(sc_ref, pacc_ref, hx_ref, bnp_ref, wp_ref, tb_ref, cr_ref,
              ox_ref, oh_ref):
    blk = sc_ref[0]
    pac = pacc_ref[0]
    aggm = blk[:, 0:16] + pac[:, 0:16]
    aggc = blk[:, 16:20] + pac[:, 16:20]
    cnt = blk[:, 20:21] + pac[:, 20:21]
    hxb = hx_ref[0]
    ox_ref[0] = hxb[:, 16:20] + jnp.where(
        cnt > 0.0, aggc / jnp.maximum(cnt, 1.0), 0.0)
    zi = _gelu(
        _bdot(hxb[:, 0:16] * bnp_ref[10:11, 0:16] + bnp_ref[11:12, 0:16],
              wp_ref[:, 48:64])
        + _bdot(aggm * bnp_ref[12:13, 0:16] + bnp_ref[13:14, 0:16],
                wp_ref[:, 64:80])
        + tb_ref[0, 0, 32:48][None, :])
    oh_ref[0] = _gelu(
        _bdot(zi * bnp_ref[14:15, 0:16] + bnp_ref[15:16, 0:16],
              wp_ref[:, 32:48]) + cr_ref[0, 16:32][None, :])


def kernel(x, h, edges, edge_weights, time_embed, message_params,
           coord_params, inv_params, Wa, ba, Wb, bb):
    del edge_weights
    s1, t1 = _bn_scale(message_params[0])
    s2, t2 = _bn_scale(message_params[1])
    sc1, tc1 = _bn_scale(coord_params[0])
    sc2, tc2 = _bn_scale(coord_params[1])
    si1, ti1 = _bn_scale(inv_params[0])
    si2, ti2 = _bn_scale(inv_params[1])
    w1, b1 = message_params[0][4], message_params[0][5]
    w2, b2 = message_params[1][4], message_params[1][5]
    wc1, bc1 = coord_params[0][4], coord_params[0][5]
    wc2, bc2 = coord_params[1][4], coord_params[1][5]
    wi1, bi1 = inv_params[0][4], inv_params[0][5]
    wi2, bi2 = inv_params[1][4], inv_params[1][5]

    def bdot(a, w):
        return jnp.dot(a.astype(_BF), w.astype(_BF),
                       preferred_element_type=f32)

    te1 = time_embed * s1[40:48] + t1[40:48]
    bias1_t = bdot(te1, w1[40:48]) + b1
    tec1 = time_embed * sc1[16:24] + tc1[16:24]
    biasc1_t = bdot(tec1, wc1[16:24]) + bc1
    tei1 = time_embed * si1[32:40] + ti1[32:40]
    biasi_t = bdot(tei1, wi1[32:40]) + bi1
    ca = bdot(time_embed, Wa[16:24]) + ba
    cb = bdot(time_embed, Wb[16:24]) + bb
    bc2_b = jnp.broadcast_to(bc2[None, :], (_B, 1))
    tb = jnp.concatenate(
        [bias1_t, biasc1_t, biasi_t, ca, cb, bc2_b,
         jnp.zeros((_B, 13), f32)], axis=1).reshape(_B, 1, 64)

    def row20(v):
        return jnp.concatenate([v, jnp.zeros((32 - v.shape[0],), f32)])
    bnp = jnp.stack([
        row20(jnp.concatenate([s1[0:16], s1[32:36]])),
        row20(jnp.concatenate([t1[0:16], t1[32:36]])),
        row20(jnp.concatenate([s1[16:32], s1[36:40]])),
        row20(jnp.concatenate([t1[16:32], t1[36:40]])),
        row20(s2), row20(t2),
        row20(sc1[0:16]), row20(tc1[0:16]),
        row20(sc2), row20(tc2),
        row20(si1[0:16]), row20(ti1[0:16]),
        row20(si1[16:32]), row20(ti1[16:32]),
        row20(si2), row20(ti2)])

    w20 = jnp.concatenate(
        [jnp.concatenate([w1[0:16], w1[32:36]], axis=0),
         jnp.concatenate([w1[16:32], w1[36:40]], axis=0)],
        axis=1).astype(_BF)
    wp = jnp.concatenate(
        [w2, wc1[0:16], wi2, wi1[0:16], wi1[16:32],
         jnp.zeros((16, 16), f32)], axis=1).astype(_BF)
    cr = jnp.concatenate([b2, bi2, jnp.zeros((32,), f32)])[None, :]
    z16 = jnp.zeros((16, 1), f32)
    wcat = jnp.concatenate(
        [jnp.concatenate([Wa[0:16], Wb[0:16], z16], axis=1),
         jnp.concatenate([z16, z16, wc2], axis=1)], axis=0)
    wcat = jnp.concatenate([wcat, jnp.zeros((32, 5), f32)],
                           axis=1).astype(_BF)

    hx = jnp.concatenate([h, x], axis=2)

    pn_t, pb_t = pl.pallas_call(
        _tab_body,
        grid=(_B,),
        in_specs=[
            pl.BlockSpec((1, _N, 20), lambda b: (b, 0, 0)),
            pl.BlockSpec((16, 32), lambda b: (0, 0)),
            pl.BlockSpec((20, 32), lambda b: (0, 0)),
            pl.BlockSpec((1, 1, 64), lambda b: (b, 0, 0)),
        ],
        out_specs=[
            pl.BlockSpec((1, _N, 16), lambda b: (b, 0, 0)),
            pl.BlockSpec((1, _N, 16), lambda b: (b, 0, 0)),
        ],
        out_shape=[
            jax.ShapeDtypeStruct((_B, _N, 16), f32),
            jax.ShapeDtypeStruct((_B, _N, 16), f32),
        ],
    )(hx, bnp, w20, tb)

    rnd = lambda w: w.astype(_BF).astype(f32)
    wsc = jnp.zeros((48, 16), f32)
    wsc = wsc.at[0:16].set(rnd(w2))
    wsc = wsc.at[16:32].set(rnd(wc1[0:16]))
    wsc = wsc.at[32].set(s2)
    wsc = wsc.at[33].set(t2)
    wsc = wsc.at[34].set(sc1[0:16])
    wsc = wsc.at[35].set(tc1[0:16])
    wsc = wsc.at[36].set(sc2)
    wsc = wsc.at[37].set(tc2)
    wsc = wsc.at[38].set(rnd(Wa[0:16, 0]))
    wsc = wsc.at[39].set(rnd(Wb[0:16, 0]))
    wsc = wsc.at[40].set(rnd(wc2[:, 0]))
    wsc = wsc.at[41].set(b2)
    tbb = jnp.concatenate(
        [biasc1_t, ca, cb, jnp.broadcast_to(bc2[None, :], (_B, 1)),
         jnp.zeros((_B, 13), f32)], axis=1)

    grid_pts = jnp.linspace(_LO, _HI, _LUT)
    gvals = _gelu(grid_pts)
    m_seg = (gvals[1:] - gvals[:-1]) / (grid_pts[1:] - grid_pts[:-1])
    c_seg = gvals[:-1] - m_seg * grid_pts[:-1]
    lutm = jnp.concatenate([m_seg, jnp.ones((1,), f32)]).astype(f32)
    lutc = jnp.concatenate([c_seg, jnp.zeros((1,), f32)]).astype(f32)

    xt = jnp.concatenate([x, jnp.zeros((_B, _N, 4), f32)],
                         axis=2).reshape(_B, _N * 8)
    pn_flat = pn_t.reshape(_B, _N * 16)
    pb_flat = pb_t.reshape(_B, _N * 16)
    nid = edges[:, :, 0].reshape(_B, _E // _CH, _CH)
    bid = edges[:, :, 1].reshape(_B, _E // _CH, _CH)
    seg = edges[0, :, 0].reshape(_E // _CH, _CH)

    mesh = plsc.VectorSubcoreMesh(core_axis_name="c", subcore_axis_name="s")
    sc_out = pl.kernel(
        _sc_body,
        mesh=mesh,
        out_type=jax.ShapeDtypeStruct((_B, _N * 32), f32),
        scratch_types=[
            pltpu.VMEM((_N * 16,), f32),
            pltpu.VMEM((_N * 16,), f32),
            pltpu.VMEM((_N * 8,), f32),
            pltpu.VMEM((_CH,), i32),
            pltpu.VMEM((_CH,), i32),
            pltpu.VMEM((_CH,), i32),
            pltpu.VMEM((_LUT,), f32),
            pltpu.VMEM((_LUT,), f32),
            pltpu.VMEM((768,), f32),
            pltpu.VMEM((32,), f32),
            pltpu.VMEM((16 * 4096,), f32),
        ],
        compiler_params=pltpu.CompilerParams(needs_layout_passes=False),
    )(pn_flat, pb_flat, xt, nid, bid, seg, lutm, lutc,
      wsc.reshape(768), tbb)

    sc_res = sc_out.reshape(_B, _N, 32)

    nidx4 = edges[:, :, 0].reshape(_B, _E // _EC, 1, _EC)
    bidx4 = (edges[:, :, 1] + _N).reshape(_B, _E // _EC, 1, _EC)
    seg3 = edges[0, :, 0].reshape(_E // _EC, 1, _EC)
    nc_tc = _E // _EC - _KT
    pacc = pl.pallas_call(
        _tcm_body,
        grid=(_B, nc_tc),
        in_specs=[
            pl.BlockSpec((1, _N, 20), lambda b, c: (b, 0, 0)),
            pl.BlockSpec((1, 1, 1, _EC), lambda b, c: (b, c + _KT, 0, 0)),
            pl.BlockSpec((1, 1, 1, _EC), lambda b, c: (b, c + _KT, 0, 0)),
            pl.BlockSpec((1, 1, _EC), lambda b, c: (c + _KT, 0, 0)),
            pl.BlockSpec((1, 1, 64), lambda b, c: (b, 0, 0)),
            pl.BlockSpec((16, 32), lambda b, c: (0, 0)),
            pl.BlockSpec((20, 32), lambda b, c: (0, 0)),
            pl.BlockSpec((16, 96), lambda b, c: (0, 0)),
            pl.BlockSpec((32, 8), lambda b, c: (0, 0)),
            pl.BlockSpec((1, 64), lambda b, c: (0, 0)),
        ],
        out_specs=[pl.BlockSpec((1, _N, 24), lambda b, c: (b, 0, 0))],
        out_shape=[jax.ShapeDtypeStruct((_B, _N, 24), f32)],
        scratch_shapes=[
            pltpu.VMEM((2 * _N, 24), _BF),
            pltpu.VMEM((2 * _N, 24), _BF),
            pltpu.VMEM((_N, 24), f32),
        ],
        compiler_params=pltpu.CompilerParams(
            dimension_semantics=("arbitrary", "arbitrary")),
    )(hx, nidx4, bidx4, seg3, tb, bnp, w20, wp, wcat, cr)[0]

    ox, oh = pl.pallas_call(
        _fin_body,
        grid=(_B,),
        in_specs=[
            pl.BlockSpec((1, _N, 32), lambda b: (b, 0, 0)),
            pl.BlockSpec((1, _N, 24), lambda b: (b, 0, 0)),
            pl.BlockSpec((1, _N, 20), lambda b: (b, 0, 0)),
            pl.BlockSpec((16, 32), lambda b: (0, 0)),
            pl.BlockSpec((16, 96), lambda b: (0, 0)),
            pl.BlockSpec((1, 1, 64), lambda b: (b, 0, 0)),
            pl.BlockSpec((1, 64), lambda b: (0, 0)),
        ],
        out_specs=[
            pl.BlockSpec((1, _N, 4), lambda b: (b, 0, 0)),
            pl.BlockSpec((1, _N, 16), lambda b: (b, 0, 0)),
        ],
        out_shape=[
            jax.ShapeDtypeStruct((_B, _N, 4), f32),
            jax.ShapeDtypeStruct((_B, _N, 16), f32),
        ],
    )(sc_res, pacc, hx, bnp, wp, tb, cr)
    return (ox, oh)
